# baseline (speedup 1.0000x reference)
import functools

import numpy as np

import jax
import jax.numpy as jnp
from jax import lax
from jax.experimental import pallas as pl
from jax.experimental.pallas import tpu as pltpu
from jax.experimental.pallas import tpu_sc as plsc

B, S, H, E, K = 4, 2048, 1024, 8, 2
CAP = (S * K) // E
CHT = 256
NW = (B * S) // CHT
NCB = S // CHT

def _router_body(hs_ref, wt_ref, probs_ref, ohb_ref, wb_ref, offs_ref,
                 aux_ref, z_ref):
    hs = hs_ref[0]
    wt = wt_ref[...]
    logits = jnp.dot(hs, wt, preferred_element_type=jnp.float32)

    m = jnp.max(logits, axis=-1, keepdims=True)
    el = jnp.exp(logits - m)
    sel = jnp.sum(el, axis=-1, keepdims=True)
    probs = el / sel
    probs_ref[0] = probs

    lse = m + jnp.log(sel)
    z_ref[...] = jnp.sum(lse * lse).reshape(1, 1, 1)
    aux_ref[...] = jnp.sum(probs * probs).reshape(1, 1, 1)

    eidx = jax.lax.broadcasted_iota(jnp.int32, (S, E), 1)
    m1 = jnp.max(probs, axis=-1, keepdims=True)
    i1 = jnp.min(jnp.where(probs == m1, eidx, E), axis=-1, keepdims=True)
    p2 = jnp.where(eidx == i1, -1.0, probs)
    m2 = jnp.max(p2, axis=-1, keepdims=True)
    i2 = jnp.min(jnp.where(p2 == m2, eidx, E), axis=-1, keepdims=True)

    wsum = m1 + m2
    w1 = m1 / wsum
    w2 = m2 / wsum

    oh1 = eidx == i1
    oh2 = eidx == i2

    ohf2 = jnp.concatenate(
        [oh1.astype(jnp.float32), oh2.astype(jnp.float32)], axis=1)
    ohb_ref[0] = ohf2
    wb_ref[0] = jnp.concatenate(
        [jnp.where(oh1, w1, 0.0), jnp.where(oh2, w2, 0.0)], axis=1)

    cs = [jnp.sum(ohf2[j * CHT:(j + 1) * CHT], axis=0, keepdims=True)
          for j in range(NCB)]
    run = jnp.zeros((1, 2 * E), jnp.float32)
    rows = []
    for j in range(NCB):
        rows.append(run)
        run = run + cs[j]
    offs = jnp.concatenate(rows, axis=0)
    offs = jnp.concatenate(
        [offs[:, :E], offs[:, E:] + run[:, :E]], axis=1)
    offs_ref[0] = offs


_sc_mesh = plsc.VectorSubcoreMesh(core_axis_name="c", subcore_axis_name="s")


@functools.partial(
    pl.kernel,
    mesh=_sc_mesh,
    out_type=[
        jax.ShapeDtypeStruct((B * S, 16), jnp.float32),
        jax.ShapeDtypeStruct((B * S, 16), jnp.float32),
    ],
    scratch_types=[
        pltpu.VMEM((CHT, 16), jnp.float32),
        pltpu.VMEM((CHT, 16), jnp.float32),
        pltpu.VMEM((1, 16), jnp.float32),
    ],
)
def _sc_dispatch(ohb_hbm, wb_hbm, offs_hbm, acc_hbm, comb_hbm,
                 ohb_v, wb_v, off_v):
    wid = lax.axis_index("s") * 2 + lax.axis_index("c")
    base = wid * CHT
    pltpu.sync_copy(ohb_hbm.at[pl.ds(base, CHT)], ohb_v)
    pltpu.sync_copy(wb_hbm.at[pl.ds(base, CHT)], wb_v)
    pltpu.sync_copy(offs_hbm.at[pl.ds(wid, 1)], off_v)

    lane = lax.iota(jnp.int32, 16)
    off = off_v[0]
    capf = jnp.float32(CAP)

    def scan_body(t, carry):
        v = ohb_v[t]
        acc = jnp.where(carry + off < capf, v, 0.0)
        ohb_v[t] = acc
        wb_v[t] = acc * wb_v[t]
        return carry + v

    lax.fori_loop(0, CHT, scan_body, lane.astype(jnp.float32) * 0.0)

    pltpu.sync_copy(ohb_v, acc_hbm.at[pl.ds(base, CHT)])
    pltpu.sync_copy(wb_v, comb_hbm.at[pl.ds(base, CHT)])


@jax.jit
def kernel(hidden_states, W):
    wt = W.T
    probs, ohb, wb, offs, aux, z = pl.pallas_call(
        _router_body,
        grid=(B,),
        in_specs=[
            pl.BlockSpec((1, S, H), lambda b: (b, 0, 0)),
            pl.BlockSpec((H, E), lambda b: (0, 0)),
        ],
        out_specs=[
            pl.BlockSpec((1, S, E), lambda b: (b, 0, 0)),
            pl.BlockSpec((1, S, 2 * E), lambda b: (b, 0, 0)),
            pl.BlockSpec((1, S, 2 * E), lambda b: (b, 0, 0)),
            pl.BlockSpec((1, NCB, 2 * E), lambda b: (b, 0, 0)),
            pl.BlockSpec((1, 1, 1), lambda b: (b, 0, 0)),
            pl.BlockSpec((1, 1, 1), lambda b: (b, 0, 0)),
        ],
        out_shape=[
            jax.ShapeDtypeStruct((B, S, E), jnp.float32),
            jax.ShapeDtypeStruct((B, S, 2 * E), jnp.float32),
            jax.ShapeDtypeStruct((B, S, 2 * E), jnp.float32),
            jax.ShapeDtypeStruct((B, NCB, 2 * E), jnp.float32),
            jax.ShapeDtypeStruct((B, 1, 1), jnp.float32),
            jax.ShapeDtypeStruct((B, 1, 1), jnp.float32),
        ],
    )(hidden_states, wt)

    a16, c16 = _sc_dispatch(
        ohb.reshape(B * S, 2 * E),
        wb.reshape(B * S, 2 * E),
        offs.reshape(B * NCB, 2 * E),
    )
    disp = (a16[:, :E] + a16[:, E:]).reshape(B, S, E)
    comb = (c16[:, :E] + c16[:, E:]).reshape(B, S, E)
    aux_loss = (jnp.sum(aux) / (B * S)) * E
    z_loss = jnp.sum(z) / (B * S)
    return (disp, comb, probs, aux_loss.reshape(()), z_loss.reshape(()))


# speedup vs baseline: 1.1107x; 1.1107x over previous
import functools

import numpy as np

import jax
import jax.numpy as jnp
from jax import lax
from jax.experimental import pallas as pl
from jax.experimental.pallas import tpu as pltpu
from jax.experimental.pallas import tpu_sc as plsc

B, S, H, E, K = 4, 2048, 1024, 8, 2
CAP = (S * K) // E
CHT = 256
NW = (B * S) // CHT
NCB = S // CHT

def _router_body(hs_ref, wt_ref, probs_ref, ohb_ref, wb_ref, offs_ref,
                 aux_ref, z_ref):
    hs = hs_ref[0]
    wt = wt_ref[...]
    logits = jnp.dot(hs, wt, preferred_element_type=jnp.float32)

    m = jnp.max(logits, axis=-1, keepdims=True)
    el = jnp.exp(logits - m)
    sel = jnp.sum(el, axis=-1, keepdims=True)
    probs = el / sel
    probs_ref[0] = probs

    lse = m + jnp.log(sel)
    z_ref[...] = jnp.sum(lse * lse).reshape(1, 1, 1)
    aux_ref[...] = jnp.sum(probs * probs).reshape(1, 1, 1)

    eidx = jax.lax.broadcasted_iota(jnp.int32, (S, E), 1)
    m1 = jnp.max(probs, axis=-1, keepdims=True)
    i1 = jnp.min(jnp.where(probs == m1, eidx, E), axis=-1, keepdims=True)
    p2 = jnp.where(eidx == i1, -1.0, probs)
    m2 = jnp.max(p2, axis=-1, keepdims=True)
    i2 = jnp.min(jnp.where(p2 == m2, eidx, E), axis=-1, keepdims=True)

    wsum = m1 + m2
    w1 = m1 / wsum
    w2 = m2 / wsum

    oh1 = eidx == i1
    oh2 = eidx == i2

    ohf2 = jnp.concatenate(
        [oh1.astype(jnp.float32), oh2.astype(jnp.float32)], axis=1)
    ohb_ref[0] = ohf2
    wb_ref[0] = jnp.concatenate(
        [jnp.where(oh1, w1, 0.0), jnp.where(oh2, w2, 0.0)], axis=1)

    cs = [jnp.sum(ohf2[j * CHT:(j + 1) * CHT], axis=0, keepdims=True)
          for j in range(NCB)]
    run = jnp.zeros((1, 2 * E), jnp.float32)
    rows = []
    for j in range(NCB):
        rows.append(run)
        run = run + cs[j]
    offs = jnp.concatenate(rows, axis=0)
    offs = jnp.concatenate(
        [offs[:, :E], offs[:, E:] + run[:, :E]], axis=1)
    offs_ref[0] = offs


_sc_mesh = plsc.VectorSubcoreMesh(core_axis_name="c", subcore_axis_name="s")


@functools.partial(
    pl.kernel,
    mesh=_sc_mesh,
    out_type=[
        jax.ShapeDtypeStruct((B * S * E // 16, 16), jnp.float32),
        jax.ShapeDtypeStruct((B * S * E // 16, 16), jnp.float32),
    ],
    scratch_types=[
        pltpu.VMEM((CHT, 16), jnp.float32),
        pltpu.VMEM((CHT, 16), jnp.float32),
        pltpu.VMEM((CHT // 2, 16), jnp.float32),
        pltpu.VMEM((CHT // 2, 16), jnp.float32),
        pltpu.VMEM((1, 16), jnp.float32),
    ],
)
def _sc_dispatch(ohb_hbm, wb_hbm, offs_hbm, disp_hbm, comb_hbm,
                 ohb_v, wb_v, disp_v, comb_v, off_v):
    wid = lax.axis_index("s") * 2 + lax.axis_index("c")
    base = wid * CHT
    pltpu.sync_copy(ohb_hbm.at[pl.ds(base, CHT)], ohb_v)
    pltpu.sync_copy(wb_hbm.at[pl.ds(base, CHT)], wb_v)
    pltpu.sync_copy(offs_hbm.at[pl.ds(wid, 1)], off_v)

    lane = lax.iota(jnp.int32, 16)
    lane_e = lax.rem(lane, 8)
    idx_hi = jnp.where(lane < 8, lane + 8, lane)
    lo = lane < 8
    off = off_v[0]
    capf = jnp.float32(CAP)

    gd = lax.GatherDimensionNumbers(
        offset_dims=(), collapsed_slice_dims=(0,), start_index_map=(0,))

    def take16(x, idx):
        return lax.gather(x, idx[:, None], gd, slice_sizes=(1,),
                          mode=lax.GatherScatterMode.PROMISE_IN_BOUNDS)

    def fold(x):
        return x + take16(x, idx_hi)

    def pair_body(t, carry):
        v0 = ohb_v[2 * t]
        v1 = ohb_v[2 * t + 1]
        a0 = jnp.where(carry + off < capf, v0, 0.0)
        carry = carry + v0
        a1 = jnp.where(carry + off < capf, v1, 0.0)
        carry = carry + v1
        c0 = a0 * wb_v[2 * t]
        c1 = a1 * wb_v[2 * t + 1]
        d0 = fold(a0)
        d1 = take16(fold(a1), lane_e)
        disp_v[t] = jnp.where(lo, d0, d1)
        e0 = fold(c0)
        e1 = take16(fold(c1), lane_e)
        comb_v[t] = jnp.where(lo, e0, e1)
        return carry

    lax.fori_loop(0, CHT // 2, pair_body, lane.astype(jnp.float32) * 0.0)

    obase = wid * (CHT // 2)
    pltpu.sync_copy(disp_v, disp_hbm.at[pl.ds(obase, CHT // 2)])
    pltpu.sync_copy(comb_v, comb_hbm.at[pl.ds(obase, CHT // 2)])


@jax.jit
def kernel(hidden_states, W):
    wt = W.T
    probs, ohb, wb, offs, aux, z = pl.pallas_call(
        _router_body,
        grid=(B,),
        in_specs=[
            pl.BlockSpec((1, S, H), lambda b: (b, 0, 0)),
            pl.BlockSpec((H, E), lambda b: (0, 0)),
        ],
        out_specs=[
            pl.BlockSpec((1, S, E), lambda b: (b, 0, 0)),
            pl.BlockSpec((1, S, 2 * E), lambda b: (b, 0, 0)),
            pl.BlockSpec((1, S, 2 * E), lambda b: (b, 0, 0)),
            pl.BlockSpec((1, NCB, 2 * E), lambda b: (b, 0, 0)),
            pl.BlockSpec((1, 1, 1), lambda b: (b, 0, 0)),
            pl.BlockSpec((1, 1, 1), lambda b: (b, 0, 0)),
        ],
        out_shape=[
            jax.ShapeDtypeStruct((B, S, E), jnp.float32),
            jax.ShapeDtypeStruct((B, S, 2 * E), jnp.float32),
            jax.ShapeDtypeStruct((B, S, 2 * E), jnp.float32),
            jax.ShapeDtypeStruct((B, NCB, 2 * E), jnp.float32),
            jax.ShapeDtypeStruct((B, 1, 1), jnp.float32),
            jax.ShapeDtypeStruct((B, 1, 1), jnp.float32),
        ],
    )(hidden_states, wt)

    d16, c16 = _sc_dispatch(
        ohb.reshape(B * S, 2 * E),
        wb.reshape(B * S, 2 * E),
        offs.reshape(B * NCB, 2 * E),
    )
    disp = d16.reshape(B, S, E)
    comb = c16.reshape(B, S, E)
    aux_loss = (jnp.sum(aux) / (B * S)) * E
    z_loss = jnp.sum(z) / (B * S)
    return (disp, comb, probs, aux_loss.reshape(()), z_loss.reshape(()))


# speedup vs baseline: 1.6105x; 1.4500x over previous
import functools

import jax
import jax.numpy as jnp
from jax.experimental import pallas as pl

B, S, H, E, K = 4, 2048, 1024, 8, 2
CAP = (S * K) // E


def _router_body(hs_ref, wt_ref, disp_ref, comb_ref, probs_ref, aux_ref, z_ref):
    hs = hs_ref[0]
    wt = wt_ref[...]
    logits = jnp.dot(hs, wt, preferred_element_type=jnp.float32)

    m = jnp.max(logits, axis=-1, keepdims=True)
    el = jnp.exp(logits - m)
    sel = jnp.sum(el, axis=-1, keepdims=True)
    probs = el / sel
    probs_ref[0] = probs

    lse = m + jnp.log(sel)
    z_ref[...] = jnp.sum(lse * lse).reshape(1, 1, 1)
    aux_ref[...] = jnp.sum(probs * probs).reshape(1, 1, 1)

    eidx = jax.lax.broadcasted_iota(jnp.int32, (S, E), 1)
    m1 = jnp.max(probs, axis=-1, keepdims=True)
    i1 = jnp.min(jnp.where(probs == m1, eidx, E), axis=-1, keepdims=True)
    p2 = jnp.where(eidx == i1, -1.0, probs)
    m2 = jnp.max(p2, axis=-1, keepdims=True)
    i2 = jnp.min(jnp.where(p2 == m2, eidx, E), axis=-1, keepdims=True)

    wsum = m1 + m2
    w1 = m1 / wsum
    w2 = m2 / wsum

    oh1 = eidx == i1
    oh2 = eidx == i2

    ohf2 = jnp.concatenate(
        [oh1.astype(jnp.float32), oh2.astype(jnp.float32)], axis=1)
    x = ohf2
    d = 1
    while d < S:
        x = x + jnp.concatenate(
            [jnp.zeros((d, 2 * E), jnp.float32), x[:S - d]], axis=0)
        d *= 2
    rank = x - ohf2
    r1 = rank[:, :E]
    r2 = rank[:, E:] + x[S - 1:, :E]

    a1 = (oh1 & (r1 < CAP)).astype(jnp.float32)
    a2 = (oh2 & (r2 < CAP)).astype(jnp.float32)
    disp_ref[0] = a1 + a2
    comb_ref[0] = a1 * w1 + a2 * w2


@functools.partial(jax.jit, static_argnames=())
def kernel(hidden_states, W):
    wt = W.T
    disp, comb, probs, aux, z = pl.pallas_call(
        _router_body,
        grid=(B,),
        in_specs=[
            pl.BlockSpec((1, S, H), lambda b: (b, 0, 0)),
            pl.BlockSpec((H, E), lambda b: (0, 0)),
        ],
        out_specs=[
            pl.BlockSpec((1, S, E), lambda b: (b, 0, 0)),
            pl.BlockSpec((1, S, E), lambda b: (b, 0, 0)),
            pl.BlockSpec((1, S, E), lambda b: (b, 0, 0)),
            pl.BlockSpec((1, 1, 1), lambda b: (b, 0, 0)),
            pl.BlockSpec((1, 1, 1), lambda b: (b, 0, 0)),
        ],
        out_shape=[
            jax.ShapeDtypeStruct((B, S, E), jnp.float32),
            jax.ShapeDtypeStruct((B, S, E), jnp.float32),
            jax.ShapeDtypeStruct((B, S, E), jnp.float32),
            jax.ShapeDtypeStruct((B, 1, 1), jnp.float32),
            jax.ShapeDtypeStruct((B, 1, 1), jnp.float32),
        ],
    )(hidden_states, wt)
    aux_loss = (jnp.sum(aux) / (B * S)) * E
    z_loss = jnp.sum(z) / (B * S)
    return (disp, comb, probs, aux_loss.reshape(()), z_loss.reshape(()))
